# Initial kernel scaffold; baseline (speedup 1.0000x reference)
import jax
import jax.numpy as jnp
from jax.experimental import pallas as pl


def kernel(atom_number, embedding_list):
    raise NotImplementedError("write your pallas kernel here")


# speedup vs baseline: 1.3327x; 1.3327x over previous
import functools

import jax
import jax.numpy as jnp
from jax import lax
from jax.experimental import pallas as pl
from jax.experimental.pallas import tpu as pltpu
from jax.experimental.pallas import tpu_sc as plsc

B = 1_000_000
D = 128
NC, NS = 2, 16
NW = NC * NS
SUP = 1600
SUB = 64
N_SUB = SUP // SUB
N_SUPERS = B // SUP
SUPERS_PER_W = (N_SUPERS + NW - 1) // NW

_mesh = plsc.VectorSubcoreMesh(core_axis_name="c", subcore_axis_name="s")


@functools.partial(
    pl.kernel,
    out_type=jax.ShapeDtypeStruct((B, D), jnp.float32),
    mesh=_mesh,
    scratch_types=[
        pltpu.VMEM((SUP,), jnp.int32),
        pltpu.VMEM((SUB, D), jnp.float32),
        pltpu.SemaphoreType.DMA,
    ],
)
def _gather_kernel(idx_hbm, table_hbm, out_hbm, idx_v, rows_v, sem):
    wid = lax.axis_index("s") * NC + lax.axis_index("c")

    def super_body(t_i, carry):
        t = t_i * NW + wid

        @pl.when(t < N_SUPERS)
        def _():
            base = t * SUP
            pltpu.sync_copy(idx_hbm.at[pl.ds(base, SUP)], idx_v)

            def sub_body(j, c):
                pltpu.async_copy(
                    table_hbm.at[idx_v.at[pl.ds(j * SUB, SUB)]], rows_v, sem
                ).wait()
                pltpu.sync_copy(rows_v, out_hbm.at[pl.ds(base + j * SUB, SUB)])
                return c

            lax.fori_loop(0, N_SUB, sub_body, 0)

        return carry

    lax.fori_loop(0, SUPERS_PER_W, super_body, 0)


def kernel(atom_number, embedding_list):
    return _gather_kernel(atom_number, embedding_list)


# speedup vs baseline: 1.3563x; 1.0178x over previous
import functools

import jax
import jax.numpy as jnp
from jax import lax
from jax.experimental import pallas as pl
from jax.experimental.pallas import tpu as pltpu
from jax.experimental.pallas import tpu_sc as plsc

B = 1_000_000
D = 128
NC, NS = 2, 16
NW = NC * NS
W = 31_248
SUB = 56
N_SUB = W // SUB
NBUF = 6
GROUPS = N_SUB // NBUF
GA = 4
SL = 2
TAIL_BASE = NW * W
TAIL = B - TAIL_BASE

_mesh = plsc.VectorSubcoreMesh(core_axis_name="c", subcore_axis_name="s")


@functools.partial(
    pl.kernel,
    out_type=jax.ShapeDtypeStruct((B, D), jnp.float32),
    mesh=_mesh,
    scratch_types=[
        pltpu.VMEM((W,), jnp.int32),
        [pltpu.VMEM((SUB, D), jnp.float32) for _ in range(NBUF)],
        [pltpu.SemaphoreType.DMA for _ in range(NBUF)],
        [pltpu.SemaphoreType.DMA for _ in range(NBUF)],
        pltpu.VMEM((TAIL,), jnp.int32),
        pltpu.VMEM((TAIL, D), jnp.float32),
        pltpu.SemaphoreType.DMA,
    ],
)
def _gather_kernel(idx_hbm, table_hbm, out_hbm, idx_v, bufs, sg, ss,
                   tidx_v, trows_v, tsem):
    wid = lax.axis_index("s") * NC + lax.axis_index("c")
    base = wid * W
    pltpu.sync_copy(idx_hbm.at[pl.ds(base, W)], idx_v)

    def g_copy(j, b):
        return pltpu.make_async_copy(
            table_hbm.at[idx_v.at[pl.ds(j * SUB, SUB)]], bufs[b], sg[b])

    def s_copy(j, b):
        return pltpu.make_async_copy(
            bufs[b], out_hbm.at[pl.ds(base + j * SUB, SUB)], ss[b])

    for j in range(GA):
        g_copy(j, j % NBUF).start()

    def step(j, jj):
        b = j % NBUF

        @pl.when(jj >= SL)
        def _():
            s_copy(jj - SL, (b - SL) % NBUF).wait()

        @pl.when(jj + GA < N_SUB)
        def _():
            g_copy(jj + GA, (b + GA) % NBUF).start()

        g_copy(jj, b).wait()
        s_copy(jj, b).start()

    def group(jo, carry):
        for b in range(NBUF):
            step(b, jo * NBUF + b)
        return carry

    lax.fori_loop(0, GROUPS, group, 0)

    for j in range(N_SUB - SL, N_SUB):
        s_copy(j, j % NBUF).wait()

    @pl.when(wid == 0)
    def _():
        pltpu.sync_copy(idx_hbm.at[pl.ds(TAIL_BASE, TAIL)], tidx_v)
        pltpu.async_copy(table_hbm.at[tidx_v], trows_v, tsem).wait()
        pltpu.sync_copy(trows_v, out_hbm.at[pl.ds(TAIL_BASE, TAIL)])


def kernel(atom_number, embedding_list):
    return _gather_kernel(atom_number, embedding_list)


# speedup vs baseline: 10.9084x; 8.0425x over previous
import functools

import jax
import jax.numpy as jnp
from jax import lax
from jax.experimental import pallas as pl
from jax.experimental.pallas import tpu as pltpu
from jax.experimental.pallas import tpu_sc as plsc

B = 1_000_000
D = 128
NC, NS = 2, 16
NW = NC * NS
W = 31_248
SUB = 56
N_SUB = W // SUB
NBUF = 6
GROUPS = N_SUB // NBUF
GA = 4
SL = 2
TAIL_BASE = NW * W
TAIL = B - TAIL_BASE

_mesh = plsc.VectorSubcoreMesh(core_axis_name="c", subcore_axis_name="s")


@functools.partial(
    pl.kernel,
    out_type=jax.ShapeDtypeStruct((B, D), jnp.float32),
    mesh=_mesh,
    scratch_types=[
        pltpu.VMEM((W,), jnp.int32),
        [pltpu.VMEM((SUB, D), jnp.float32) for _ in range(NBUF)],
        [pltpu.SemaphoreType.DMA for _ in range(NBUF)],
        [pltpu.SemaphoreType.DMA for _ in range(NBUF)],
        pltpu.VMEM((TAIL,), jnp.int32),
        pltpu.VMEM((TAIL, D), jnp.float32),
        pltpu.SemaphoreType.DMA,
        pltpu.VMEM_SHARED((83, D), jnp.float32),
    ],
)
def _gather_kernel(idx_hbm, table_hbm, out_hbm, idx_v, bufs, sg, ss,
                   tidx_v, trows_v, tsem, table_sh):
    wid = lax.axis_index("s") * NC + lax.axis_index("c")
    base = wid * W

    @pl.when(lax.axis_index("s") == 0)
    def _():
        pltpu.sync_copy(table_hbm, table_sh)

    plsc.subcore_barrier()

    pltpu.sync_copy(idx_hbm.at[pl.ds(base, W)], idx_v)

    def g_copy(j, b):
        return pltpu.make_async_copy(
            table_sh.at[idx_v.at[pl.ds(j * SUB, SUB)]], bufs[b], sg[b])

    def s_copy(j, b):
        return pltpu.make_async_copy(
            bufs[b], out_hbm.at[pl.ds(base + j * SUB, SUB)], ss[b])

    for j in range(GA):
        g_copy(j, j % NBUF).start()

    def step(j, jj):
        b = j % NBUF

        @pl.when(jj >= SL)
        def _():
            s_copy(jj - SL, (b - SL) % NBUF).wait()

        @pl.when(jj + GA < N_SUB)
        def _():
            g_copy(jj + GA, (b + GA) % NBUF).start()

        g_copy(jj, b).wait()
        s_copy(jj, b).start()

    def group(jo, carry):
        for b in range(NBUF):
            step(b, jo * NBUF + b)
        return carry

    lax.fori_loop(0, GROUPS, group, 0)

    for j in range(N_SUB - SL, N_SUB):
        s_copy(j, j % NBUF).wait()

    @pl.when(wid == 0)
    def _():
        pltpu.sync_copy(idx_hbm.at[pl.ds(TAIL_BASE, TAIL)], tidx_v)
        pltpu.async_copy(table_sh.at[tidx_v], trows_v, tsem).wait()
        pltpu.sync_copy(trows_v, out_hbm.at[pl.ds(TAIL_BASE, TAIL)])


def kernel(atom_number, embedding_list):
    return _gather_kernel(atom_number, embedding_list)
